# baseline (speedup 1.0000x reference)
import functools
import math

import jax
import jax.numpy as jnp
from jax import lax
from jax.experimental import pallas as pl
from jax.experimental.pallas import tpu as pltpu
from jax.experimental.pallas import tpu_sc as plsc

NC = 2
NS = 16
NW = NC * NS
LANES = 16

CHUNK = 128
NBUF = 4


@functools.cache
def _build_relayout(n_vh: int, d: int):
    mesh = plsc.VectorSubcoreMesh(core_axis_name="c", subcore_axis_name="s")
    dh = d // 8
    PADD = d + 1

    scratch = (
        [pltpu.VMEM((dh, 8, CHUNK), jnp.float32) for _ in range(NBUF)]
        + [pltpu.VMEM((CHUNK, PADD), jnp.float32) for _ in range(NBUF)]
        + [pltpu.SemaphoreType.DMA for _ in range(2 * NBUF)]
    )

    @functools.partial(
        pl.kernel,
        out_type=jax.ShapeDtypeStruct((n_vh * CHUNK, d), jnp.float32),
        mesh=mesh,
        scratch_types=scratch,
        compiler_params=pltpu.CompilerParams(
            use_tc_tiling_on_sc=False, needs_layout_passes=False
        ),
    )
    def rly(t4_hbm, out_hbm, *scr):
        ibuf = scr[:NBUF]
        obuf = scr[NBUF:2 * NBUF]
        isem = scr[2 * NBUF:3 * NBUF]
        osem = scr[3 * NBUF:4 * NBUF]

        w = lax.axis_index("c") * NS + lax.axis_index("s")

        lane = lax.iota(jnp.int32, LANES)
        vrow = [vlb * LANES + lane for vlb in range(CHUNK // LANES)]

        n_my = n_vh // NW

        def start_in(k, b):
            vh = k * NW + w
            pltpu.make_async_copy(t4_hbm.at[:, vh], ibuf[b], isem[b]).start()

        def transpose(b):
            @plsc.parallel_loop(0, d, 1, unroll=8)
            def _t(dd):
                dvec = jnp.full((LANES,), dd, jnp.int32)
                hi = dd // 8
                lo = dd % 8
                for vlb in range(CHUNK // LANES):
                    v = ibuf[b][hi, lo, pl.ds(vlb * LANES, LANES)]
                    plsc.store_scatter(obuf[b], [vrow[vlb], dvec], v)

        def out_dst(k, b):
            vh = k * NW + w
            return pltpu.make_async_copy(
                obuf[b].at[:, pl.ds(0, d)],
                out_hbm.at[pl.ds(vh * CHUNK, CHUNK)],
                osem[b],
            )

        for b in range(NBUF):
            start_in(b, b)

        def outer(o, _):
            for b in range(NBUF):
                k = o * NBUF + b
                vh = k * NW + w
                pltpu.make_async_copy(
                    t4_hbm.at[:, vh], ibuf[b], isem[b]
                ).wait()

                @pl.when(o > 0)
                def _():
                    out_dst(k, b).wait()

                transpose(b)
                out_dst(k, b).start()

                @pl.when(o < (n_my // NBUF) - 1)
                def _():
                    start_in(k + NBUF, b)

            return 0

        lax.fori_loop(0, n_my // NBUF, outer, 0)

        for b in range(NBUF):
            out_dst((n_my // NBUF - 1) * NBUF + b, b).wait()

    return rly


@functools.cache
def _build_gather(n_s: int, d: int, n_rows: int):
    mesh = plsc.VectorSubcoreMesh(core_axis_name="c", subcore_axis_name="s")
    dh = d // 8
    ngrp = d // LANES
    PADC = CHUNK + 1

    scratch = (
        [pltpu.VMEM((n_s, CHUNK), jnp.int32)]
        + [pltpu.VMEM((CHUNK, d), jnp.float32) for _ in range(NBUF)]
        + [pltpu.VMEM((dh, 8, PADC), jnp.float32) for _ in range(NBUF)]
        + [pltpu.SemaphoreType.DMA for _ in range(2 * NBUF)]
    )

    @functools.partial(
        pl.kernel,
        out_type=jax.ShapeDtypeStruct((n_s, dh, NW, 8, CHUNK), jnp.float32),
        mesh=mesh,
        scratch_types=scratch,
        compiler_params=pltpu.CompilerParams(
            use_tc_tiling_on_sc=False, needs_layout_passes=False
        ),
    )
    def emb(table_hbm, idx_hbm, out_hbm, *scr):
        idx_v = scr[0]
        gbuf = scr[1:1 + NBUF]
        obuf = scr[1 + NBUF:1 + 2 * NBUF]
        gsem = scr[1 + 2 * NBUF:1 + 3 * NBUF]
        osem = scr[1 + 3 * NBUF:1 + 4 * NBUF]

        w = lax.axis_index("c") * NS + lax.axis_index("s")

        pltpu.sync_copy(idx_hbm.at[:, pl.ds(w * CHUNK, CHUNK)], idx_v)

        lane = lax.iota(jnp.int32, LANES)
        hi_idx = [(g * LANES + lane) // 8 for g in range(ngrp)]
        mid_idx = [(g * LANES + lane) % 8 for g in range(ngrp)]

        def start_gather(s, b):
            pltpu.make_async_copy(
                table_hbm.at[idx_v.at[s]], gbuf[b], gsem[b]
            ).start()

        def transpose_scale(b):
            @plsc.parallel_loop(0, CHUNK, 1, unroll=4)
            def _ts(r):
                rvec = jnp.full((LANES,), r, jnp.int32)
                for g in range(ngrp):
                    v = gbuf[b][r, pl.ds(g * LANES, LANES)]
                    plsc.store_scatter(
                        obuf[b], [hi_idx[g], mid_idx[g], rvec], v * 8.0
                    )

        n_outer = n_s // NBUF

        for b in range(NBUF):
            start_gather(b, b)

        def outer(o, _):
            for b in range(NBUF):
                s = o * NBUF + b
                pltpu.make_async_copy(
                    table_hbm.at[idx_v.at[s]], gbuf[b], gsem[b]
                ).wait()

                @pl.when(o > 0)
                def _():
                    pltpu.make_async_copy(
                        obuf[b].at[:, :, pl.ds(0, CHUNK)],
                        out_hbm.at[s, :, w], osem[b]
                    ).wait()

                transpose_scale(b)
                pltpu.make_async_copy(
                    obuf[b].at[:, :, pl.ds(0, CHUNK)],
                    out_hbm.at[s, :, w], osem[b]
                ).start()

                @pl.when(o < n_outer - 1)
                def _():
                    start_gather(s + NBUF, b)

            return 0

        lax.fori_loop(0, n_outer, outer, 0)

        for b in range(NBUF):
            pltpu.make_async_copy(
                obuf[b].at[:, :, pl.ds(0, CHUNK)],
                out_hbm.at[(n_outer - 1) * NBUF + b, :, w], osem[b]
            ).wait()

    return emb


def kernel(x, table):
    d = table.shape[1]
    v = table.shape[0]
    q, n_s = x.shape
    assert q == NW * CHUNK and d % 8 == 0
    idx_t = jnp.transpose(x).astype(jnp.int32)

    n_vh = -(-v // CHUNK)
    n_vh += (-n_vh) % (NW * NBUF)
    v_pad = n_vh * CHUNK

    t_pad = jnp.pad(table, ((0, v_pad - v), (0, 0)))
    t4 = t_pad.reshape(n_vh, CHUNK, d // 8, 8)
    t4 = jnp.transpose(t4, (2, 0, 3, 1))

    t_lin = _build_relayout(n_vh, d)(t4)
    out5 = _build_gather(n_s, d, v_pad)(t_lin, idx_t)
    return jnp.transpose(out5, (2, 4, 0, 1, 3)).reshape(q, n_s, d)


# speedup vs baseline: 1.0893x; 1.0893x over previous
import functools
import math

import jax
import jax.numpy as jnp
from jax import lax
from jax.experimental import pallas as pl
from jax.experimental.pallas import tpu as pltpu
from jax.experimental.pallas import tpu_sc as plsc

NC = 2
NS = 16
NW = NC * NS
LANES = 16

CHUNK = 128
NBUF = 4


@functools.cache
def _build_relayout(n_vh: int, d: int):
    mesh = plsc.VectorSubcoreMesh(core_axis_name="c", subcore_axis_name="s")
    dh = d // 8
    PADD = d + 1

    scratch = (
        [pltpu.VMEM((dh, 8, CHUNK), jnp.float32) for _ in range(NBUF)]
        + [pltpu.VMEM((CHUNK, PADD), jnp.float32) for _ in range(NBUF)]
        + [pltpu.SemaphoreType.DMA for _ in range(2 * NBUF)]
    )

    @functools.partial(
        pl.kernel,
        out_type=jax.ShapeDtypeStruct((n_vh * CHUNK, d), jnp.float32),
        mesh=mesh,
        scratch_types=scratch,
        compiler_params=pltpu.CompilerParams(
            use_tc_tiling_on_sc=False, needs_layout_passes=False
        ),
    )
    def rly(t4_hbm, out_hbm, *scr):
        ibuf = scr[:NBUF]
        obuf = scr[NBUF:2 * NBUF]
        isem = scr[2 * NBUF:3 * NBUF]
        osem = scr[3 * NBUF:4 * NBUF]

        w = lax.axis_index("c") * NS + lax.axis_index("s")

        lane = lax.iota(jnp.int32, LANES)
        vrow = [vlb * LANES + lane for vlb in range(CHUNK // LANES)]

        n_my = n_vh // NW

        def start_in(k, b):
            vh = k * NW + w
            pltpu.make_async_copy(t4_hbm.at[:, vh], ibuf[b], isem[b]).start()

        def transpose(b):
            @plsc.parallel_loop(0, d, 1, unroll=4)
            def _t(dd):
                dvec = jnp.full((LANES,), dd, jnp.int32)
                hi = dd // 8
                lo = dd % 8
                for vlb in range(CHUNK // LANES):
                    v = ibuf[b][hi, lo, pl.ds(vlb * LANES, LANES)]
                    plsc.store_scatter(obuf[b], [vrow[vlb], dvec], v)

        def out_dst(k, b):
            vh = k * NW + w
            return pltpu.make_async_copy(
                obuf[b].at[:, pl.ds(0, d)],
                out_hbm.at[pl.ds(vh * CHUNK, CHUNK)],
                osem[b],
            )

        for b in range(NBUF):
            start_in(b, b)

        def outer(o, _):
            for b in range(NBUF):
                k = o * NBUF + b
                vh = k * NW + w
                pltpu.make_async_copy(
                    t4_hbm.at[:, vh], ibuf[b], isem[b]
                ).wait()

                @pl.when(o > 0)
                def _():
                    out_dst(k, b).wait()

                transpose(b)
                out_dst(k, b).start()

                @pl.when(o < (n_my // NBUF) - 1)
                def _():
                    start_in(k + NBUF, b)

            return 0

        lax.fori_loop(0, n_my // NBUF, outer, 0)

        for b in range(NBUF):
            out_dst((n_my // NBUF - 1) * NBUF + b, b).wait()

    return rly


@functools.cache
def _build_gather(n_s: int, d: int, n_rows: int):
    mesh = plsc.VectorSubcoreMesh(core_axis_name="c", subcore_axis_name="s")
    dh = d // 8
    ngrp = d // LANES
    PADC = CHUNK + 1

    scratch = (
        [pltpu.VMEM((n_s, CHUNK), jnp.int32)]
        + [pltpu.VMEM((CHUNK, d), jnp.float32) for _ in range(NBUF)]
        + [pltpu.VMEM((dh, 8, PADC), jnp.float32) for _ in range(NBUF)]
        + [pltpu.SemaphoreType.DMA for _ in range(2 * NBUF)]
    )

    @functools.partial(
        pl.kernel,
        out_type=jax.ShapeDtypeStruct((n_s, dh, NW, 8, CHUNK), jnp.float32),
        mesh=mesh,
        scratch_types=scratch,
        compiler_params=pltpu.CompilerParams(
            use_tc_tiling_on_sc=False, needs_layout_passes=False
        ),
    )
    def emb(table_hbm, idx_hbm, out_hbm, *scr):
        idx_v = scr[0]
        gbuf = scr[1:1 + NBUF]
        obuf = scr[1 + NBUF:1 + 2 * NBUF]
        gsem = scr[1 + 2 * NBUF:1 + 3 * NBUF]
        osem = scr[1 + 3 * NBUF:1 + 4 * NBUF]

        w = lax.axis_index("c") * NS + lax.axis_index("s")

        pltpu.sync_copy(idx_hbm.at[:, pl.ds(w * CHUNK, CHUNK)], idx_v)

        lane = lax.iota(jnp.int32, LANES)
        hi_idx = [(g * LANES + lane) // 8 for g in range(ngrp)]
        mid_idx = [(g * LANES + lane) % 8 for g in range(ngrp)]

        def start_gather(s, b):
            pltpu.make_async_copy(
                table_hbm.at[idx_v.at[s]], gbuf[b], gsem[b]
            ).start()

        def transpose_scale(b):
            @plsc.parallel_loop(0, CHUNK, 1, unroll=4)
            def _ts(r):
                rvec = jnp.full((LANES,), r, jnp.int32)
                for g in range(ngrp):
                    v = gbuf[b][r, pl.ds(g * LANES, LANES)]
                    plsc.store_scatter(
                        obuf[b], [hi_idx[g], mid_idx[g], rvec], v * 8.0
                    )

        n_outer = n_s // NBUF

        for b in range(NBUF):
            start_gather(b, b)

        def outer(o, _):
            for b in range(NBUF):
                s = o * NBUF + b
                pltpu.make_async_copy(
                    table_hbm.at[idx_v.at[s]], gbuf[b], gsem[b]
                ).wait()

                @pl.when(o > 0)
                def _():
                    pltpu.make_async_copy(
                        obuf[b].at[:, :, pl.ds(0, CHUNK)],
                        out_hbm.at[s, :, w], osem[b]
                    ).wait()

                transpose_scale(b)
                pltpu.make_async_copy(
                    obuf[b].at[:, :, pl.ds(0, CHUNK)],
                    out_hbm.at[s, :, w], osem[b]
                ).start()

                @pl.when(o < n_outer - 1)
                def _():
                    start_gather(s + NBUF, b)

            return 0

        lax.fori_loop(0, n_outer, outer, 0)

        for b in range(NBUF):
            pltpu.make_async_copy(
                obuf[b].at[:, :, pl.ds(0, CHUNK)],
                out_hbm.at[(n_outer - 1) * NBUF + b, :, w], osem[b]
            ).wait()

    return emb


def kernel(x, table):
    d = table.shape[1]
    v = table.shape[0]
    q, n_s = x.shape
    assert q == NW * CHUNK and d % 8 == 0
    idx_t = jnp.transpose(x).astype(jnp.int32)

    n_vh = -(-v // CHUNK)
    n_vh += (-n_vh) % (NW * NBUF)
    v_pad = n_vh * CHUNK

    t_pad = jnp.pad(table, ((0, v_pad - v), (0, 0)))
    t4 = t_pad.reshape(n_vh, CHUNK, d // 8, 8)
    t4 = jnp.transpose(t4, (2, 0, 3, 1))

    t_lin = _build_relayout(n_vh, d)(t4)
    out5 = _build_gather(n_s, d, v_pad)(t_lin, idx_t)
    return jnp.transpose(out5, (2, 4, 0, 1, 3)).reshape(q, n_s, d)
